# baseline (speedup 1.0000x reference)
import functools

import jax
import jax.numpy as jnp
from jax import lax
from jax.experimental import pallas as pl
from jax.experimental.pallas import tpu as pltpu
from jax.experimental.pallas import tpu_sc as plsc

_NC = 2
_NS = 16
_CH = 80
_NBUF = 4


def _sc_edge_aggregate(ei3, x, n_nodes, d, cht, npad):
    mesh = plsc.VectorSubcoreMesh(
        core_axis_name="c", subcore_axis_name="s",
        num_cores=_NC, num_subcores=_NS)
    zpt = npad // _NS
    rpt = (n_nodes // _NS) // 8 * 8
    rem = n_nodes - rpt * _NS

    ib = 8
    nsup = cht // ib
    assert cht % ib == 0

    @functools.partial(
        pl.kernel,
        out_type=jax.ShapeDtypeStruct((_NC, n_nodes, d), jnp.float32),
        mesh=mesh,
        scratch_types=[
            pltpu.VMEM((2 * ib, _CH), jnp.int32),
            pltpu.VMEM((2 * ib, _CH), jnp.int32),
            [pltpu.VMEM((_CH, d), jnp.float32)] * _NBUF,
            pltpu.VMEM_SHARED((npad, d), jnp.float32),
            pltpu.SemaphoreType.DMA,
            pltpu.SemaphoreType.DMA,
            pltpu.SemaphoreType.DMA,
        ],
    )
    def k(ei_hbm, x_hbm, out_hbm, src_v, dst_v, bufs,
          agg_sh, gsem, ssem, isem):
        src_hbm = ei_hbm.at[0]
        dst_hbm = ei_hbm.at[1]
        c = lax.axis_index("c")
        s = lax.axis_index("s")
        wid = c * _NS + s
        row0 = wid * cht
        pltpu.sync_copy(src_hbm.at[pl.ds(row0, ib)], src_v.at[pl.ds(0, ib)])
        pltpu.sync_copy(dst_hbm.at[pl.ds(row0, ib)], dst_v.at[pl.ds(0, ib)])
        pltpu.async_copy(x_hbm.at[src_v.at[0]], bufs[0], gsem)
        pltpu.async_copy(x_hbm.at[src_v.at[1]], bufs[1], gsem)

        zbuf = bufs[2]
        zero16 = jnp.zeros((16,), jnp.float32)

        def zrow(r, carry):
            for q in range(d // 16):
                zbuf[r, pl.ds(q * 16, 16)] = zero16
            return carry

        lax.fori_loop(0, _CH, zrow, 0)
        for t in range(zpt // _CH):
            pltpu.sync_copy(zbuf, agg_sh.at[pl.ds(s * zpt + t * _CH, _CH)])
        plsc.subcore_barrier()

        def sup(i, carry):
            base = i * ib
            for kk in range(ib):
                j = base + kk
                r = j % (2 * ib)
                buf = bufs[kk % _NBUF]
                nbuf2 = bufs[(kk + 2) % _NBUF]
                @pl.when(j >= 2)
                def _drain():
                    pltpu.make_async_copy(nbuf2, agg_sh.at[dst_v.at[r]], ssem).wait()
                if kk == 1:
                    @pl.when(i + 1 < nsup)
                    def _prefetch():
                        off = row0 + (i + 1) * ib
                        slot = ((i + 1) % 2) * ib
                        pltpu.async_copy(src_hbm.at[pl.ds(off, ib)],
                                         src_v.at[pl.ds(slot, ib)], isem)
                        pltpu.async_copy(dst_hbm.at[pl.ds(off, ib)],
                                         dst_v.at[pl.ds(slot, ib)], isem)
                if kk == ib - 2:
                    @pl.when(i + 1 < nsup)
                    def _idx_ready():
                        pltpu.make_async_copy(src_hbm.at[pl.ds(row0, ib)],
                                              src_v.at[pl.ds(0, ib)], isem).wait()
                        pltpu.make_async_copy(dst_hbm.at[pl.ds(row0, ib)],
                                              dst_v.at[pl.ds(0, ib)], isem).wait()
                @pl.when(j + 2 < cht)
                def _next_gather():
                    r2 = (j + 2) % (2 * ib)
                    pltpu.async_copy(x_hbm.at[src_v.at[r2]], nbuf2, gsem)
                pltpu.make_async_copy(x_hbm.at[src_v.at[r]], buf, gsem).wait()
                pltpu.async_copy(buf, agg_sh.at[dst_v.at[r]], ssem, add=True)
            return carry

        lax.fori_loop(0, nsup, sup, 0)
        pltpu.make_async_copy(bufs[0], agg_sh.at[dst_v.at[0]], ssem).wait()
        pltpu.make_async_copy(bufs[1], agg_sh.at[dst_v.at[0]], ssem).wait()
        plsc.subcore_barrier()
        pltpu.sync_copy(agg_sh.at[pl.ds(s * rpt, rpt)],
                        out_hbm.at[c, pl.ds(s * rpt, rpt)])
        if rem:
            @pl.when(s == _NS - 1)
            def _tail():
                pltpu.sync_copy(agg_sh.at[pl.ds(rpt * _NS, rem)],
                                out_hbm.at[c, pl.ds(rpt * _NS, rem)])

    return k(ei3, x)


def _tc_root_body(x_ref, wrootT, brel, xr_ref):
    xr_ref[...] = jnp.dot(x_ref[...], wrootT[...],
                          preferred_element_type=jnp.float32) + brel[...]


def _tc_body(agg_ref, xr_ref, batch_ref, w1T, b1, w2T, b2, wrelT,
             out_ref, pooled, *, nblocks, bk):
    i = pl.program_id(0)

    @pl.when(i == 0)
    def _init():
        pooled[...] = jnp.full(pooled.shape, -jnp.inf, jnp.float32)

    a = agg_ref[0] + agg_ref[1]
    h = jnp.dot(a, wrelT[...], preferred_element_type=jnp.float32)
    h = jnp.maximum(h + xr_ref[...], 0.0)

    bcol = batch_ref[...]
    g_lo = batch_ref[0, 0]
    g_hi = batch_ref[bk - 1, 0]

    def upd(gi, carry):
        m = jnp.where(bcol == gi, h, -jnp.inf)
        mx = jnp.max(m, axis=0)
        pooled[pl.ds(gi, 1), :] = jnp.maximum(pooled[pl.ds(gi, 1), :], mx[None, :])
        return carry

    lax.fori_loop(g_lo, g_hi + 1, upd, 0)

    @pl.when(i == nblocks - 1)
    def _final():
        p = pooled[...]
        h1 = jnp.dot(p, w1T[...], preferred_element_type=jnp.float32)
        h1 = jnp.maximum(h1 + b1[...], 0.0)
        out_ref[...] = jnp.dot(h1, w2T[...], preferred_element_type=jnp.float32) + b2[...]


def kernel(x, edge_index, batch, W_rel, b_rel, W_root, W1, b1, W2, b2):
    n, d = x.shape
    e = edge_index.shape[1]
    g = 64
    ghc = W_rel.shape[0]
    nhid = W1.shape[0]
    nout = W2.shape[0]

    nw = _NC * _NS
    cht = -(-e // (nw * _CH))
    cht += (-cht) % 8
    ep = nw * cht * _CH
    npad = n + (-n) % (_NS * _CH)

    pad = ep - e
    pad_idx = jnp.arange(pad, dtype=jnp.int32)
    padvals = jnp.stack([pad_idx % n, n + pad_idx % (npad - n)])
    assert e % _CH == 0
    ei3 = jnp.concatenate([edge_index.reshape(2, e // _CH, _CH),
                           padvals.reshape(2, pad // _CH, _CH)], axis=1)

    bk = 2000
    nblocks = n // bk
    batch2 = batch.reshape(n, 1)

    xr = pl.pallas_call(
        _tc_root_body,
        grid=(nblocks,),
        in_specs=[
            pl.BlockSpec((bk, d), lambda i: (i, 0)),
            pl.BlockSpec((d, ghc), lambda i: (0, 0)),
            pl.BlockSpec((1, ghc), lambda i: (0, 0)),
        ],
        out_specs=pl.BlockSpec((bk, ghc), lambda i: (i, 0)),
        out_shape=jax.ShapeDtypeStruct((n, ghc), jnp.float32),
        compiler_params=pltpu.CompilerParams(
            dimension_semantics=("arbitrary",)),
    )(x, W_root.T, b_rel.reshape(1, -1))

    agg = _sc_edge_aggregate(ei3, x, n, d, cht, npad)

    out = pl.pallas_call(
        functools.partial(_tc_body, nblocks=nblocks, bk=bk),
        grid=(nblocks,),
        in_specs=[
            pl.BlockSpec((_NC, bk, d), lambda i: (0, i, 0)),
            pl.BlockSpec((bk, ghc), lambda i: (i, 0)),
            pl.BlockSpec((bk, 1), lambda i: (i, 0)),
            pl.BlockSpec((ghc, nhid), lambda i: (0, 0)),
            pl.BlockSpec((1, nhid), lambda i: (0, 0)),
            pl.BlockSpec((nhid, nout), lambda i: (0, 0)),
            pl.BlockSpec((1, nout), lambda i: (0, 0)),
            pl.BlockSpec((d, ghc), lambda i: (0, 0)),
        ],
        out_specs=pl.BlockSpec((g, nout), lambda i: (0, 0)),
        out_shape=jax.ShapeDtypeStruct((g, nout), jnp.float32),
        scratch_shapes=[pltpu.VMEM((g, ghc), jnp.float32)],
        compiler_params=pltpu.CompilerParams(
            dimension_semantics=("arbitrary",)),
    )(agg, xr, batch2, W1.T, b1.reshape(1, -1), W2.T, b2.reshape(1, -1), W_rel.T)
    return out


# speedup vs baseline: 1.0642x; 1.0642x over previous
import functools

import jax
import jax.numpy as jnp
from jax import lax
from jax.experimental import pallas as pl
from jax.experimental.pallas import tpu as pltpu
from jax.experimental.pallas import tpu_sc as plsc

_NC = 2
_NS = 16
_CH = 80
_NBUF = 4


def _sc_edge_aggregate(ei3, x, n_nodes, d, cht, npad):
    mesh = plsc.VectorSubcoreMesh(
        core_axis_name="c", subcore_axis_name="s",
        num_cores=_NC, num_subcores=_NS)
    zpt = npad // _NS
    rpt = (n_nodes // _NS) // 8 * 8
    rem = n_nodes - rpt * _NS

    ib = 8
    nsup = cht // ib
    assert cht % ib == 0

    @functools.partial(
        pl.kernel,
        out_type=jax.ShapeDtypeStruct((_NC, n_nodes, d), jnp.float32),
        mesh=mesh,
        scratch_types=[
            pltpu.VMEM((2 * ib, _CH), jnp.int32),
            pltpu.VMEM((2 * ib, _CH), jnp.int32),
            [pltpu.VMEM((_CH, d), jnp.float32)] * _NBUF,
            pltpu.VMEM_SHARED((npad, d), jnp.float32),
            pltpu.SemaphoreType.DMA,
            pltpu.SemaphoreType.DMA,
            pltpu.SemaphoreType.DMA,
        ],
    )
    def k(ei_hbm, x_hbm, out_hbm, src_v, dst_v, bufs,
          agg_sh, gsem, ssem, isem):
        src_hbm = ei_hbm.at[0]
        dst_hbm = ei_hbm.at[1]
        c = lax.axis_index("c")
        s = lax.axis_index("s")
        wid = c * _NS + s
        row0 = wid * cht
        pltpu.sync_copy(src_hbm.at[pl.ds(row0, ib)], src_v.at[pl.ds(0, ib)])
        pltpu.sync_copy(dst_hbm.at[pl.ds(row0, ib)], dst_v.at[pl.ds(0, ib)])
        pltpu.async_copy(x_hbm.at[src_v.at[0]], bufs[0], gsem)
        pltpu.async_copy(x_hbm.at[src_v.at[1]], bufs[1], gsem)

        zbuf = bufs[2]
        zero16 = jnp.zeros((16,), jnp.float32)

        def zrow(r, carry):
            for q in range(d // 16):
                zbuf[r, pl.ds(q * 16, 16)] = zero16
            return carry

        lax.fori_loop(0, _CH, zrow, 0)
        for t in range(zpt // _CH):
            pltpu.sync_copy(zbuf, agg_sh.at[pl.ds(s * zpt + t * _CH, _CH)])
        plsc.subcore_barrier()

        def sup(i, carry):
            base = i * ib
            for kk in range(ib):
                j = base + kk
                r = j % (2 * ib)
                buf = bufs[kk % _NBUF]
                nbuf2 = bufs[(kk + 2) % _NBUF]
                @pl.when(j >= 2)
                def _drain():
                    pltpu.make_async_copy(nbuf2, agg_sh.at[dst_v.at[r]], ssem).wait()
                if kk == 1:
                    @pl.when(i + 1 < nsup)
                    def _prefetch():
                        off = row0 + (i + 1) * ib
                        slot = ((i + 1) % 2) * ib
                        pltpu.async_copy(src_hbm.at[pl.ds(off, ib)],
                                         src_v.at[pl.ds(slot, ib)], isem)
                        pltpu.async_copy(dst_hbm.at[pl.ds(off, ib)],
                                         dst_v.at[pl.ds(slot, ib)], isem)
                if kk == ib - 2:
                    @pl.when(i + 1 < nsup)
                    def _idx_ready():
                        pltpu.make_async_copy(src_hbm.at[pl.ds(row0, ib)],
                                              src_v.at[pl.ds(0, ib)], isem).wait()
                        pltpu.make_async_copy(dst_hbm.at[pl.ds(row0, ib)],
                                              dst_v.at[pl.ds(0, ib)], isem).wait()
                @pl.when(j + 2 < cht)
                def _next_gather():
                    r2 = (j + 2) % (2 * ib)
                    pltpu.async_copy(x_hbm.at[src_v.at[r2]], nbuf2, gsem)
                pltpu.make_async_copy(x_hbm.at[src_v.at[r]], buf, gsem).wait()
                pltpu.async_copy(buf, agg_sh.at[dst_v.at[r]], ssem, add=True)
            return carry

        lax.fori_loop(0, nsup, sup, 0)
        pltpu.make_async_copy(bufs[0], agg_sh.at[dst_v.at[0]], ssem).wait()
        pltpu.make_async_copy(bufs[1], agg_sh.at[dst_v.at[0]], ssem).wait()
        plsc.subcore_barrier()
        pltpu.sync_copy(agg_sh.at[pl.ds(s * rpt, rpt)],
                        out_hbm.at[c, pl.ds(s * rpt, rpt)])
        if rem:
            @pl.when(s == _NS - 1)
            def _tail():
                pltpu.sync_copy(agg_sh.at[pl.ds(rpt * _NS, rem)],
                                out_hbm.at[c, pl.ds(rpt * _NS, rem)])

    return k(ei3, x)


def _tc_root_body(x_ref, wrootT, brel, xr_ref):
    xr_ref[...] = jnp.dot(x_ref[...], wrootT[...],
                          preferred_element_type=jnp.float32) + brel[...]


def _tc_body(agg_ref, xr_ref, batch_ref, w1T, b1, w2T, b2, wrelT,
             out_ref, pooled, *, nblocks, bk):
    i = pl.program_id(0)

    @pl.when(i == 0)
    def _init():
        pooled[...] = jnp.full(pooled.shape, -jnp.inf, jnp.float32)

    a = agg_ref[0] + agg_ref[1]
    h = jnp.dot(a, wrelT[...], preferred_element_type=jnp.float32)
    h = jnp.maximum(h + xr_ref[...], 0.0)

    bcol = batch_ref[...]
    g_lo = batch_ref[0, 0]
    g_hi = batch_ref[bk - 1, 0]

    def upd(gi, carry):
        m = jnp.where(bcol == gi, h, -jnp.inf)
        mx = jnp.max(m, axis=0)
        pooled[pl.ds(gi, 1), :] = jnp.maximum(pooled[pl.ds(gi, 1), :], mx[None, :])
        return carry

    lax.fori_loop(g_lo, g_hi + 1, upd, 0)

    @pl.when(i == nblocks - 1)
    def _final():
        p = pooled[...]
        h1 = jnp.dot(p, w1T[...], preferred_element_type=jnp.float32)
        h1 = jnp.maximum(h1 + b1[...], 0.0)
        out_ref[...] = jnp.dot(h1, w2T[...], preferred_element_type=jnp.float32) + b2[...]


def kernel(x, edge_index, batch, W_rel, b_rel, W_root, W1, b1, W2, b2):
    n, d = x.shape
    e = edge_index.shape[1]
    g = 64
    ghc = W_rel.shape[0]
    nhid = W1.shape[0]
    nout = W2.shape[0]

    nw = _NC * _NS
    cht = -(-e // (nw * _CH))
    cht += (-cht) % 8
    ep = nw * cht * _CH
    npad = n + (-n) % (_NS * _CH)

    pad = ep - e
    pad_idx = jnp.arange(pad, dtype=jnp.int32)
    padvals = jnp.stack([pad_idx % n, n + pad_idx % (npad - n)])
    assert e % _CH == 0
    ei3 = jnp.concatenate([edge_index.reshape(2, e // _CH, _CH),
                           padvals.reshape(2, pad // _CH, _CH)], axis=1)

    bk = 1000
    nblocks = n // bk
    batch2 = batch.reshape(n, 1)

    xr = pl.pallas_call(
        _tc_root_body,
        grid=(nblocks,),
        in_specs=[
            pl.BlockSpec((bk, d), lambda i: (i, 0)),
            pl.BlockSpec((d, ghc), lambda i: (0, 0)),
            pl.BlockSpec((1, ghc), lambda i: (0, 0)),
        ],
        out_specs=pl.BlockSpec((bk, ghc), lambda i: (i, 0)),
        out_shape=jax.ShapeDtypeStruct((n, ghc), jnp.float32),
        compiler_params=pltpu.CompilerParams(
            dimension_semantics=("arbitrary",)),
    )(x, W_root.T, b_rel.reshape(1, -1))

    agg = _sc_edge_aggregate(ei3, x, n, d, cht, npad)

    out = pl.pallas_call(
        functools.partial(_tc_body, nblocks=nblocks, bk=bk),
        grid=(nblocks,),
        in_specs=[
            pl.BlockSpec((_NC, bk, d), lambda i: (0, i, 0)),
            pl.BlockSpec((bk, ghc), lambda i: (i, 0)),
            pl.BlockSpec((bk, 1), lambda i: (i, 0)),
            pl.BlockSpec((ghc, nhid), lambda i: (0, 0)),
            pl.BlockSpec((1, nhid), lambda i: (0, 0)),
            pl.BlockSpec((nhid, nout), lambda i: (0, 0)),
            pl.BlockSpec((1, nout), lambda i: (0, 0)),
            pl.BlockSpec((d, ghc), lambda i: (0, 0)),
        ],
        out_specs=pl.BlockSpec((g, nout), lambda i: (0, 0)),
        out_shape=jax.ShapeDtypeStruct((g, nout), jnp.float32),
        scratch_shapes=[pltpu.VMEM((g, ghc), jnp.float32)],
        compiler_params=pltpu.CompilerParams(
            dimension_semantics=("arbitrary",)),
    )(agg, xr, batch2, W1.T, b1.reshape(1, -1), W2.T, b2.reshape(1, -1), W_rel.T)
    return out
